# Initial kernel scaffold; baseline (speedup 1.0000x reference)
import functools

import jax
import jax.numpy as jnp
from jax import lax
from jax.experimental import pallas as pl
from jax.experimental.pallas import tpu as pltpu
from jax.experimental.pallas import tpu_sc as plsc

NC = 2
NS = 16
NW = NC * NS

B = 16384 * 50
D = 64
BPW = B // NW
CH = 128
NCH = BPW // CH
NBUF = 4


def _tile_body(pe_hbm, idx_hbm, out_hbm, idx_v, rows_v, gsem):
  wid = lax.axis_index("s") * NC + lax.axis_index("c")
  base = wid * BPW

  pltpu.sync_copy(idx_hbm.at[wid], idx_v)

  def start(ch, b):
    pltpu.async_copy(pe_hbm.at[idx_v.at[ch]], rows_v.at[b], gsem)

  def finish(ch, b):
    pltpu.make_async_copy(pe_hbm.at[idx_v.at[ch]], rows_v.at[b], gsem).wait()
    pltpu.sync_copy(rows_v.at[b], out_hbm.at[pl.ds(base + ch * CH, CH)])

  for b in range(NBUF):
    start(b, b)

  @pl.loop(0, NCH // NBUF - 1)
  def _(g):
    for b in range(NBUF):
      ch = g * NBUF + b
      finish(ch, b)
      start(ch + NBUF, b)

  for b in range(NBUF):
    finish(NCH - NBUF + b, b)


@functools.partial(
    pl.kernel,
    out_type=jax.ShapeDtypeStruct((B, D), jnp.float32),
    mesh=plsc.VectorSubcoreMesh(
        core_axis_name="c", subcore_axis_name="s",
        num_cores=NC, num_subcores=NS),
    scratch_types=[
        pltpu.VMEM((NCH, CH), jnp.int32),
        pltpu.VMEM((NBUF, CH, D), jnp.float32),
        pltpu.SemaphoreType.DMA,
    ],
)
def _gather(pe_hbm, idx_hbm, out_hbm, idx_v, rows_v, gsem):
  _tile_body(pe_hbm, idx_hbm, out_hbm, idx_v, rows_v, gsem)


def kernel(m, pe):
  idx = m.reshape(NW, NCH, CH).astype(jnp.int32)
  out = _gather(pe, idx)
  return out.reshape(m.shape[0], m.shape[1], D)


# speedup vs baseline: 6.2431x; 6.2431x over previous
import functools

import jax
import jax.numpy as jnp
from jax import lax
from jax.experimental import pallas as pl
from jax.experimental.pallas import tpu as pltpu
from jax.experimental.pallas import tpu_sc as plsc

NC = 2
NS = 16
NW = NC * NS

B = 16384 * 50
D = 64
BPW = B // NW
CH = 128
NCH = BPW // CH
NBUF = 4


def _tile_body(pe_hbm, idx_hbm, out_hbm, idx_v, rows_v, gsem):
  wid = lax.axis_index("s") * NC + lax.axis_index("c")
  base = wid * BPW

  pltpu.sync_copy(idx_hbm.at[wid], idx_v)

  def start(ch, b):
    pltpu.async_copy(pe_hbm.at[idx_v.at[ch]], rows_v.at[b], gsem)

  def finish(ch, b):
    pltpu.make_async_copy(pe_hbm.at[idx_v.at[ch]], rows_v.at[b], gsem).wait()
    pltpu.sync_copy(rows_v.at[b], out_hbm.at[pl.ds(base + ch * CH, CH)])

  for b in range(NBUF):
    start(b, b)

  @pl.loop(0, NCH // NBUF - 1)
  def _(g):
    for b in range(NBUF):
      ch = g * NBUF + b
      finish(ch, b)
      start(ch + NBUF, b)

  for b in range(NBUF):
    finish(NCH - NBUF + b, b)


@functools.partial(
    pl.kernel,
    out_type=jax.ShapeDtypeStruct((B, D), jnp.float32),
    mesh=plsc.VectorSubcoreMesh(
        core_axis_name="c", subcore_axis_name="s",
        num_cores=NC, num_subcores=NS),
    scratch_types=[
        pltpu.VMEM((NCH, CH), jnp.int32),
        pltpu.VMEM((NBUF, CH, D), jnp.float32),
        pltpu.SemaphoreType.DMA,
    ],
    compiler_params=pltpu.CompilerParams(use_tc_tiling_on_sc=False),
)
def _gather(pe_hbm, idx_hbm, out_hbm, idx_v, rows_v, gsem):
  _tile_body(pe_hbm, idx_hbm, out_hbm, idx_v, rows_v, gsem)


def kernel(m, pe):
  idx = m.reshape(NW, NCH, CH).astype(jnp.int32)
  out = _gather(pe, idx)
  return out.reshape(m.shape[0], m.shape[1], D)


# speedup vs baseline: 6.2637x; 1.0033x over previous
import functools

import jax
import jax.numpy as jnp
from jax import lax
from jax.experimental import pallas as pl
from jax.experimental.pallas import tpu as pltpu
from jax.experimental.pallas import tpu_sc as plsc

NC = 2
NS = 16
NW = NC * NS

B = 16384 * 50
D = 64
BPW = B // NW
CH = 128
NCH = BPW // CH
NBUF = 8
NIF = 5


def _tile_body(pe_hbm, idx_hbm, out_hbm, idx_v, rows_v, gsem):
  wid = lax.axis_index("s") * NC + lax.axis_index("c")
  base = wid * BPW

  pltpu.sync_copy(idx_hbm.at[wid], idx_v)

  def start(ch):
    pltpu.async_copy(pe_hbm.at[idx_v.at[ch]], rows_v.at[ch % NBUF], gsem)

  def wait_gather(ch):
    pltpu.make_async_copy(
        pe_hbm.at[idx_v.at[ch]], rows_v.at[ch % NBUF], gsem).wait()

  def drain(ch):
    pltpu.sync_copy(rows_v.at[ch % NBUF], out_hbm.at[pl.ds(base + ch * CH, CH)])

  for ch in range(NIF):
    start(ch)

  @pl.loop(0, NCH // NBUF - 1)
  def _(g):
    for b in range(NBUF):
      ch = g * NBUF + b
      wait_gather(ch)
      start(ch + NIF)
      drain(ch)

  for b in range(NBUF):
    ch = NCH - NBUF + b
    wait_gather(ch)
    if b < NBUF - NIF:
      start(ch + NIF)
    drain(ch)


@functools.partial(
    pl.kernel,
    out_type=jax.ShapeDtypeStruct((B, D), jnp.float32),
    mesh=plsc.VectorSubcoreMesh(
        core_axis_name="c", subcore_axis_name="s",
        num_cores=NC, num_subcores=NS),
    scratch_types=[
        pltpu.VMEM((NCH, CH), jnp.int32),
        pltpu.VMEM((NBUF, CH, D), jnp.float32),
        pltpu.SemaphoreType.DMA,
    ],
    compiler_params=pltpu.CompilerParams(use_tc_tiling_on_sc=False),
)
def _gather(pe_hbm, idx_hbm, out_hbm, idx_v, rows_v, gsem):
  _tile_body(pe_hbm, idx_hbm, out_hbm, idx_v, rows_v, gsem)


def kernel(m, pe):
  idx = m.reshape(NW, NCH, CH).astype(jnp.int32)
  out = _gather(pe, idx)
  return out.reshape(m.shape[0], m.shape[1], D)
